# Initial kernel scaffold; baseline (speedup 1.0000x reference)
import jax
import jax.numpy as jnp
from jax.experimental import pallas as pl


def kernel(x, tta, embedding):
    raise NotImplementedError("write your pallas kernel here")


# speedup vs baseline: 1.9987x; 1.9987x over previous
import functools

import jax
import jax.numpy as jnp
from jax import lax
from jax.experimental import pallas as pl
from jax.experimental.pallas import tpu as pltpu
from jax.experimental.pallas import tpu_sc as plsc

CHUNK = 128
LANES = 16


def _tta_kernel(n_rows, dim, vocab, num_cores, num_subcores):
    n_workers = num_cores * num_subcores
    per_w = n_rows // n_workers
    n_chunks = per_w // CHUNK
    mesh = plsc.VectorSubcoreMesh(core_axis_name="c", subcore_axis_name="s")

    @functools.partial(
        pl.kernel,
        mesh=mesh,
        out_type=jax.ShapeDtypeStruct((n_rows, dim), jnp.float32),
        compiler_params=pltpu.CompilerParams(use_tc_tiling_on_sc=False),
        scratch_types=[
            pltpu.VMEM((CHUNK,), jnp.int32),
            pltpu.VMEM((CHUNK, dim), jnp.float32),
            pltpu.VMEM((CHUNK, dim), jnp.float32),
            pltpu.SemaphoreType.DMA,
        ],
    )
    def k(x_hbm, idx_hbm, tab_hbm, out_hbm, idx_v, acc_v, xb_v, sem):
        wid = lax.axis_index("s") * num_cores + lax.axis_index("c")
        base = wid * per_w

        def chunk_body(c, carry):
            row0 = base + c * CHUNK
            pltpu.sync_copy(idx_hbm.at[pl.ds(row0, CHUNK)], idx_v)
            for j in range(CHUNK // LANES):
                t = idx_v[pl.ds(j * LANES, LANES)] - 1
                t = jnp.where(t < 0, t + vocab, t)
                idx_v[pl.ds(j * LANES, LANES)] = t
            gather = pltpu.async_copy(tab_hbm.at[idx_v], acc_v, sem)
            pltpu.sync_copy(x_hbm.at[pl.ds(row0, CHUNK)], xb_v)
            gather.wait()

            def add_rows(i, carry2):
                r = i * 4
                for rr in range(4):
                    for j in range(dim // LANES):
                        plsc.addupdate(
                            acc_v.at[r + rr, pl.ds(j * LANES, LANES)],
                            xb_v[r + rr, pl.ds(j * LANES, LANES)],
                        )
                return carry2

            lax.fori_loop(0, CHUNK // 4, add_rows, 0, unroll=False)
            pltpu.sync_copy(acc_v, out_hbm.at[pl.ds(row0, CHUNK)])
            return carry

        lax.fori_loop(0, n_chunks, chunk_body, 0, unroll=False)

    return k


def kernel(x, tta, embedding):
    b, h, d = x.shape
    vocab = embedding.shape[0]
    n_rows = b * h
    x2 = x.reshape(n_rows, d)
    idx = tta.reshape(n_rows).astype(jnp.int32)
    info = plsc.get_sparse_core_info()
    k = _tta_kernel(n_rows, d, vocab, info.num_cores, info.num_subcores)
    out = k(x2, idx, embedding)
    return out.reshape(b, h, d)
